# baseline (speedup 1.0000x reference)
import functools

import jax
import jax.numpy as jnp
from jax import lax
from jax.experimental import pallas as pl
from jax.experimental.pallas import tpu as pltpu
from jax.experimental.pallas import tpu_sc as plsc

NUM_ITEM = 1000000
NUM_FACTOR = 64
BATCH = 4096
TARGET_LEN = 20
NEG_LEN = 100
SEQ = TARGET_LEN + NEG_LEN
TOTAL = BATCH * SEQ

NC = 2
NS = 16
NW = NC * NS


CBLK = 16384
HSHIFT = 30 * CBLK
NTBLK = 32
NPACK = NTBLK * CBLK


def _table_tc_body(lo_ref, hi_ref, s1_ref):
    s1_ref[:, :NUM_FACTOR] = lo_ref[...].T
    s1_ref[:, NUM_FACTOR:] = hi_ref[...].T


_table_tc = pl.pallas_call(
    _table_tc_body,
    grid=(NTBLK,),
    in_specs=[
        pl.BlockSpec((NUM_FACTOR, CBLK), lambda i: (0, i)),
        pl.BlockSpec((NUM_FACTOR, CBLK), lambda i: (0, i + 30)),
    ],
    out_specs=pl.BlockSpec((CBLK, 128), lambda i: (i, 0)),
    out_shape=jax.ShapeDtypeStruct((NPACK, 128), jnp.float32),
)


G = SEQ
ROWS_W = BATCH // NW
NBUF = 4
NSTEP = ROWS_W // NBUF

_mesh = plsc.VectorSubcoreMesh(
    core_axis_name="c", subcore_axis_name="s", num_cores=NC, num_subcores=NS)


@functools.partial(
    pl.kernel,
    out_type=jax.ShapeDtypeStruct((BATCH, SEQ, NUM_FACTOR), jnp.float32),
    mesh=_mesh,
    scratch_types=[
        pltpu.VMEM((ROWS_W, G), jnp.int32),
        pltpu.VMEM((NBUF, G, NUM_FACTOR), jnp.float32),
        pltpu.SemaphoreType.DMA((NBUF,)),
    ],
    compiler_params=pltpu.CompilerParams(use_tc_tiling_on_sc=False),
)
def _gather_sc(table_hbm, idx_hbm, out_hbm, idx_v, rows_v, gsems):
    wid = lax.axis_index("s") * NC + lax.axis_index("c")
    base = wid * ROWS_W

    pltpu.sync_copy(idx_hbm.at[wid], idx_v)

    def fire(g, b):
        pltpu.async_copy(table_hbm.at[idx_v.at[g]], rows_v.at[b], gsems.at[b])

    def wait_store(g, b):
        pltpu.make_async_copy(
            table_hbm.at[idx_v.at[g]], rows_v.at[b], gsems.at[b]).wait()
        pltpu.sync_copy(rows_v.at[b], out_hbm.at[base + g])

    for b in range(NBUF):
        fire(b, b)

    def outer(s, _):
        for b in range(NBUF):
            g = s * NBUF + b
            wait_store(g, b)
            fire(g + NBUF, b)
        return _

    lax.fori_loop(0, NSTEP - 1, outer, None)
    for b in range(NBUF):
        wait_store((NSTEP - 1) * NBUF + b, b)


NTB = BATCH // 128
OUT5_SHAPE = (SEQ, 8, NTB, 8, 128)


def _out_tc_body(in_ref, out_ref):
    x3 = in_ref[...].reshape(128, 60, 128)
    for s2 in range(SEQ // 2):
        xt = x3[:, s2, :].T
        out_ref[2 * s2] = xt[:NUM_FACTOR].reshape(8, 1, 8, 128)
        out_ref[2 * s2 + 1] = xt[NUM_FACTOR:].reshape(8, 1, 8, 128)


_out_tc = pl.pallas_call(
    _out_tc_body,
    grid=(NTB,),
    in_specs=[pl.BlockSpec((SEQ // 2 * 128, 128), lambda i: (i, 0))],
    out_specs=pl.BlockSpec((SEQ, 8, 1, 8, 128), lambda i: (0, 0, i, 0, 0)),
    out_shape=jax.ShapeDtypeStruct(OUT5_SHAPE, jnp.float32),
)


def kernel(target_item_batch, neg_item_batch, W2):
    target = target_item_batch.reshape(BATCH, TARGET_LEN)
    neg = neg_item_batch.reshape(BATCH, NEG_LEN)
    item_batch = jnp.concatenate([target, neg], axis=1)
    idx0 = item_batch.astype(jnp.int32)
    idx2 = jnp.where(idx0 < HSHIFT, 2 * idx0, 2 * (idx0 - HSHIFT) + 1)
    idx = idx2.reshape(NW, ROWS_W, G)

    wt = W2.T
    s1 = _table_tc(wt, wt)
    table = s1.reshape(2 * NPACK, NUM_FACTOR)
    emb = _gather_sc(table, idx)
    out5 = _out_tc(emb.reshape(TOTAL // 2, 128))
    item_embedding = out5.transpose(2, 4, 0, 1, 3).reshape(
        BATCH, SEQ, NUM_FACTOR)
    return (item_batch, item_embedding)
